# baseline (speedup 1.0000x reference)
import functools
from itertools import combinations

import jax
import jax.numpy as jnp
import numpy as np
from jax import lax
from jax.experimental import pallas as pl
from jax.experimental.pallas import tpu as pltpu
from jax.experimental.pallas import tpu_sc as plsc

FIELDS = 26
FPAD = 32
VOCAB = 1000
EMB = 128
BATCH = 16384
DENSE = 13
HID = 512
NPAIRS = 325

SLICES = (2048, 6144, 6144, 2048)
NWORKERS = 32
CHUNK = 128

_PAIRS_NP = np.array(list(combinations(range(FIELDS), 2)), dtype=np.int32)


def _sc_gather(tables_flat, idx3):
    nchunk = idx3.shape[1]
    rows_w = nchunk * CHUNK
    mesh = plsc.VectorSubcoreMesh(core_axis_name="c", subcore_axis_name="s")

    @functools.partial(
        pl.kernel,
        mesh=mesh,
        out_type=jax.ShapeDtypeStruct((NWORKERS * rows_w, EMB), jnp.float32),
        scratch_types=[
            pltpu.VMEM((nchunk, CHUNK), jnp.int32),
            pltpu.VMEM((CHUNK, EMB), jnp.float32),
            pltpu.VMEM((CHUNK, EMB), jnp.float32),
            pltpu.SemaphoreType.DMA,
            pltpu.SemaphoreType.DMA,
        ],
    )
    def k(tab_hbm, idx_hbm, out_hbm, idx_v, rows_a, rows_b, sem_a, sem_b):
        wid = lax.axis_index("s") * 2 + lax.axis_index("c")
        pltpu.sync_copy(idx_hbm.at[wid], idx_v)
        base = wid * rows_w

        pltpu.async_copy(tab_hbm.at[idx_v.at[0]], rows_a, sem_a)

        def body(j, _):
            buf_j = jnp.remainder(j, 2)

            @pl.when(jnp.logical_and(j + 1 < nchunk, buf_j == 0))
            def _():
                pltpu.async_copy(tab_hbm.at[idx_v.at[j + 1]], rows_b, sem_b)

            @pl.when(jnp.logical_and(j + 1 < nchunk, buf_j == 1))
            def _():
                pltpu.async_copy(tab_hbm.at[idx_v.at[j + 1]], rows_a, sem_a)

            @pl.when(buf_j == 0)
            def _():
                pltpu.make_async_copy(tab_hbm.at[idx_v.at[0]], rows_a, sem_a).wait()
                pltpu.sync_copy(rows_a, out_hbm.at[pl.ds(base + j * CHUNK, CHUNK)])

            @pl.when(buf_j == 1)
            def _():
                pltpu.make_async_copy(tab_hbm.at[idx_v.at[0]], rows_b, sem_b).wait()
                pltpu.sync_copy(rows_b, out_hbm.at[pl.ds(base + j * CHUNK, CHUNK)])

            return 0

        lax.fori_loop(0, nchunk, body, 0)

    return k(tables_flat, idx3)


def _tc_compute(embs3, xv, w1g, w1x, b1, wout_t, bout):
    BT = 256
    bs = xv.shape[0]
    grid = (bs // BT,)

    def body(e_ref, xv_ref, w1g_ref, w1x_ref, b1_ref, wout_ref, bout_ref, o_ref):
        e = e_ref[...].reshape(BT, FIELDS, EMB)
        gram = lax.dot_general(
            e, e, (((2,), (2,)), ((0,), (0,))),
            preferred_element_type=jnp.float32,
        )
        g = gram.reshape(BT, FIELDS * FIELDS)
        h = (
            jnp.dot(g, w1g_ref[...], preferred_element_type=jnp.float32)
            + jnp.dot(xv_ref[...], w1x_ref[...], preferred_element_type=jnp.float32)
            + b1_ref[...]
        )
        h = jnp.maximum(h, 0.0)
        o_ref[...] = (
            jnp.dot(h, wout_ref[...], preferred_element_type=jnp.float32)
            + bout_ref[...]
        )

    return pl.pallas_call(
        body,
        grid=grid,
        in_specs=[
            pl.BlockSpec((BT * FIELDS, EMB), lambda i: (i, 0)),
            pl.BlockSpec((BT, DENSE), lambda i: (i, 0)),
            pl.BlockSpec((FIELDS * FIELDS, HID), lambda i: (0, 0)),
            pl.BlockSpec((DENSE, HID), lambda i: (0, 0)),
            pl.BlockSpec((1, HID), lambda i: (0, 0)),
            pl.BlockSpec((HID, 1), lambda i: (0, 0)),
            pl.BlockSpec((1, 1), lambda i: (0, 0)),
        ],
        out_specs=pl.BlockSpec((BT, 1), lambda i: (i, 0)),
        out_shape=jax.ShapeDtypeStruct((bs, 1), jnp.float32),
    )(embs3, xv, w1g, w1x, b1, wout_t, bout)


def kernel(xv, xi, emb_tables, W1, b1, Wout, bout):
    tables_flat = emb_tables.reshape(FIELDS * VOCAB, EMB)
    idx = xi + jnp.arange(FIELDS, dtype=jnp.int32)[None, :] * VOCAB
    idx_flat = idx.reshape(BATCH * FIELDS)

    ii = jnp.asarray(_PAIRS_NP[:, 0])
    jj = jnp.asarray(_PAIRS_NP[:, 1])
    w1g = (
        jnp.zeros((FIELDS * FIELDS, HID), jnp.float32)
        .at[ii * FIELDS + jj]
        .set(W1[:, :NPAIRS].T)
    )
    w1x = W1[:, NPAIRS:].T
    b1r = b1.reshape(1, HID)
    wout_t = Wout.T
    boutr = bout.reshape(1, 1)

    outs = []
    off = 0
    for bs in SLICES:
        nchunk = bs * FIELDS // (NWORKERS * CHUNK)
        idx3 = idx_flat[off * FIELDS : (off + bs) * FIELDS].reshape(
            NWORKERS, nchunk, CHUNK
        )
        embs3 = _sc_gather(tables_flat, idx3)
        outs.append(
            _tc_compute(embs3, xv[off : off + bs], w1g, w1x, b1r, wout_t, boutr)
        )
        off += bs
    return jnp.concatenate(outs, axis=0)


# speedup vs baseline: 1.0202x; 1.0202x over previous
import functools
from itertools import combinations

import jax
import jax.numpy as jnp
import numpy as np
from jax import lax
from jax.experimental import pallas as pl
from jax.experimental.pallas import tpu as pltpu
from jax.experimental.pallas import tpu_sc as plsc

FIELDS = 26
FPAD = 32
VOCAB = 1000
EMB = 128
BATCH = 16384
DENSE = 13
HID = 512
NPAIRS = 325

SLICES = (2048, 4096, 4096, 4096, 2048)
NWORKERS = 32
CHUNK = 128

_PAIRS_NP = np.array(list(combinations(range(FIELDS), 2)), dtype=np.int32)


def _sc_gather(tables_flat, idx3):
    nchunk = idx3.shape[1]
    rows_w = nchunk * CHUNK
    mesh = plsc.VectorSubcoreMesh(core_axis_name="c", subcore_axis_name="s")

    @functools.partial(
        pl.kernel,
        mesh=mesh,
        out_type=jax.ShapeDtypeStruct((NWORKERS * rows_w, EMB), jnp.float32),
        scratch_types=[
            pltpu.VMEM((nchunk, CHUNK), jnp.int32),
            pltpu.VMEM((CHUNK, EMB), jnp.float32),
            pltpu.VMEM((CHUNK, EMB), jnp.float32),
            pltpu.SemaphoreType.DMA,
            pltpu.SemaphoreType.DMA,
        ],
    )
    def k(tab_hbm, idx_hbm, out_hbm, idx_v, rows_a, rows_b, sem_a, sem_b):
        wid = lax.axis_index("s") * 2 + lax.axis_index("c")
        pltpu.sync_copy(idx_hbm.at[wid], idx_v)
        base = wid * rows_w

        pltpu.async_copy(tab_hbm.at[idx_v.at[0]], rows_a, sem_a)

        def body(j, _):
            buf_j = jnp.remainder(j, 2)

            @pl.when(jnp.logical_and(j + 1 < nchunk, buf_j == 0))
            def _():
                pltpu.async_copy(tab_hbm.at[idx_v.at[j + 1]], rows_b, sem_b)

            @pl.when(jnp.logical_and(j + 1 < nchunk, buf_j == 1))
            def _():
                pltpu.async_copy(tab_hbm.at[idx_v.at[j + 1]], rows_a, sem_a)

            @pl.when(buf_j == 0)
            def _():
                pltpu.make_async_copy(tab_hbm.at[idx_v.at[0]], rows_a, sem_a).wait()
                pltpu.sync_copy(rows_a, out_hbm.at[pl.ds(base + j * CHUNK, CHUNK)])

            @pl.when(buf_j == 1)
            def _():
                pltpu.make_async_copy(tab_hbm.at[idx_v.at[0]], rows_b, sem_b).wait()
                pltpu.sync_copy(rows_b, out_hbm.at[pl.ds(base + j * CHUNK, CHUNK)])

            return 0

        lax.fori_loop(0, nchunk, body, 0)

    return k(tables_flat, idx3)


def _tc_compute(embs3, xv, w1g, w1x, b1, wout_t, bout):
    BT = 256
    bs = xv.shape[0]
    grid = (bs // BT,)

    def body(e_ref, xv_ref, w1g_ref, w1x_ref, b1_ref, wout_ref, bout_ref, o_ref):
        e = e_ref[...].reshape(BT, FIELDS, EMB)
        gram = lax.dot_general(
            e, e, (((2,), (2,)), ((0,), (0,))),
            preferred_element_type=jnp.float32,
        )
        g = gram.reshape(BT, FIELDS * FIELDS)
        h = (
            jnp.dot(g, w1g_ref[...], preferred_element_type=jnp.float32)
            + jnp.dot(xv_ref[...], w1x_ref[...], preferred_element_type=jnp.float32)
            + b1_ref[...]
        )
        h = jnp.maximum(h, 0.0)
        o_ref[...] = (
            jnp.dot(h, wout_ref[...], preferred_element_type=jnp.float32)
            + bout_ref[...]
        )

    return pl.pallas_call(
        body,
        grid=grid,
        in_specs=[
            pl.BlockSpec((BT * FIELDS, EMB), lambda i: (i, 0)),
            pl.BlockSpec((BT, DENSE), lambda i: (i, 0)),
            pl.BlockSpec((FIELDS * FIELDS, HID), lambda i: (0, 0)),
            pl.BlockSpec((DENSE, HID), lambda i: (0, 0)),
            pl.BlockSpec((1, HID), lambda i: (0, 0)),
            pl.BlockSpec((HID, 1), lambda i: (0, 0)),
            pl.BlockSpec((1, 1), lambda i: (0, 0)),
        ],
        out_specs=pl.BlockSpec((BT, 1), lambda i: (i, 0)),
        out_shape=jax.ShapeDtypeStruct((bs, 1), jnp.float32),
    )(embs3, xv, w1g, w1x, b1, wout_t, bout)


def kernel(xv, xi, emb_tables, W1, b1, Wout, bout):
    tables_flat = emb_tables.reshape(FIELDS * VOCAB, EMB)
    idx = xi + jnp.arange(FIELDS, dtype=jnp.int32)[None, :] * VOCAB
    idx_flat = idx.reshape(BATCH * FIELDS)

    ii = jnp.asarray(_PAIRS_NP[:, 0])
    jj = jnp.asarray(_PAIRS_NP[:, 1])
    w1g = (
        jnp.zeros((FIELDS * FIELDS, HID), jnp.float32)
        .at[ii * FIELDS + jj]
        .set(W1[:, :NPAIRS].T)
    )
    w1x = W1[:, NPAIRS:].T
    b1r = b1.reshape(1, HID)
    wout_t = Wout.T
    boutr = bout.reshape(1, 1)

    outs = []
    off = 0
    for bs in SLICES:
        nchunk = bs * FIELDS // (NWORKERS * CHUNK)
        idx3 = idx_flat[off * FIELDS : (off + bs) * FIELDS].reshape(
            NWORKERS, nchunk, CHUNK
        )
        embs3 = _sc_gather(tables_flat, idx3)
        outs.append(
            _tc_compute(embs3, xv[off : off + bs], w1g, w1x, b1r, wout_t, boutr)
        )
        off += bs
    return jnp.concatenate(outs, axis=0)


# speedup vs baseline: 1.0773x; 1.0559x over previous
import functools
from itertools import combinations

import jax
import jax.numpy as jnp
import numpy as np
from jax import lax
from jax.experimental import pallas as pl
from jax.experimental.pallas import tpu as pltpu
from jax.experimental.pallas import tpu_sc as plsc

FIELDS = 26
FPAD = 32
VOCAB = 1000
EMB = 128
BATCH = 16384
DENSE = 13
HID = 512
NPAIRS = 325

SLICES = (4096, 4096, 4096, 4096)
NWORKERS = 32
CHUNK = 128

_PAIRS_NP = np.array(list(combinations(range(FIELDS), 2)), dtype=np.int32)


def _sc_gather(tables_flat, idx3):
    nchunk = idx3.shape[1]
    rows_w = nchunk * CHUNK
    mesh = plsc.VectorSubcoreMesh(core_axis_name="c", subcore_axis_name="s")

    @functools.partial(
        pl.kernel,
        mesh=mesh,
        out_type=jax.ShapeDtypeStruct((NWORKERS * rows_w, EMB), jnp.float32),
        scratch_types=[
            pltpu.VMEM((nchunk, CHUNK), jnp.int32),
            pltpu.VMEM((CHUNK, EMB), jnp.float32),
            pltpu.VMEM((CHUNK, EMB), jnp.float32),
            pltpu.SemaphoreType.DMA,
            pltpu.SemaphoreType.DMA,
        ],
    )
    def k(tab_hbm, idx_hbm, out_hbm, idx_v, rows_a, rows_b, sem_a, sem_b):
        wid = lax.axis_index("s") * 2 + lax.axis_index("c")
        pltpu.sync_copy(idx_hbm.at[wid], idx_v)
        base = wid * rows_w

        pltpu.async_copy(tab_hbm.at[idx_v.at[0]], rows_a, sem_a)

        def body(j, _):
            buf_j = jnp.remainder(j, 2)

            @pl.when(jnp.logical_and(j + 1 < nchunk, buf_j == 0))
            def _():
                pltpu.async_copy(tab_hbm.at[idx_v.at[j + 1]], rows_b, sem_b)

            @pl.when(jnp.logical_and(j + 1 < nchunk, buf_j == 1))
            def _():
                pltpu.async_copy(tab_hbm.at[idx_v.at[j + 1]], rows_a, sem_a)

            @pl.when(buf_j == 0)
            def _():
                pltpu.make_async_copy(tab_hbm.at[idx_v.at[0]], rows_a, sem_a).wait()
                pltpu.sync_copy(rows_a, out_hbm.at[pl.ds(base + j * CHUNK, CHUNK)])

            @pl.when(buf_j == 1)
            def _():
                pltpu.make_async_copy(tab_hbm.at[idx_v.at[0]], rows_b, sem_b).wait()
                pltpu.sync_copy(rows_b, out_hbm.at[pl.ds(base + j * CHUNK, CHUNK)])

            return 0

        lax.fori_loop(0, nchunk, body, 0)

    return k(tables_flat, idx3)


def _tc_compute(embs3, xv, w1g, w1x, b1, wout_t, bout):
    BT = 512
    bs = xv.shape[0]
    grid = (bs // BT,)

    def body(e_ref, xv_ref, w1g_ref, w1x_ref, b1_ref, wout_ref, bout_ref, o_ref):
        e = e_ref[...].reshape(BT, FIELDS, EMB)
        gram = lax.dot_general(
            e, e, (((2,), (2,)), ((0,), (0,))),
            preferred_element_type=jnp.float32,
        )
        g = gram.reshape(BT, FIELDS * FIELDS)
        h = (
            jnp.dot(g, w1g_ref[...], preferred_element_type=jnp.float32)
            + jnp.dot(xv_ref[...], w1x_ref[...], preferred_element_type=jnp.float32)
            + b1_ref[...]
        )
        h = jnp.maximum(h, 0.0)
        o_ref[...] = (
            jnp.dot(h, wout_ref[...], preferred_element_type=jnp.float32)
            + bout_ref[...]
        )

    return pl.pallas_call(
        body,
        grid=grid,
        in_specs=[
            pl.BlockSpec((BT * FIELDS, EMB), lambda i: (i, 0)),
            pl.BlockSpec((BT, DENSE), lambda i: (i, 0)),
            pl.BlockSpec((FIELDS * FIELDS, HID), lambda i: (0, 0)),
            pl.BlockSpec((DENSE, HID), lambda i: (0, 0)),
            pl.BlockSpec((1, HID), lambda i: (0, 0)),
            pl.BlockSpec((HID, 1), lambda i: (0, 0)),
            pl.BlockSpec((1, 1), lambda i: (0, 0)),
        ],
        out_specs=pl.BlockSpec((BT, 1), lambda i: (i, 0)),
        out_shape=jax.ShapeDtypeStruct((bs, 1), jnp.float32),
    )(embs3, xv, w1g, w1x, b1, wout_t, bout)


def kernel(xv, xi, emb_tables, W1, b1, Wout, bout):
    tables_flat = emb_tables.reshape(FIELDS * VOCAB, EMB)
    idx = xi + jnp.arange(FIELDS, dtype=jnp.int32)[None, :] * VOCAB
    idx_flat = idx.reshape(BATCH * FIELDS)

    ii = jnp.asarray(_PAIRS_NP[:, 0])
    jj = jnp.asarray(_PAIRS_NP[:, 1])
    w1g = (
        jnp.zeros((FIELDS * FIELDS, HID), jnp.float32)
        .at[ii * FIELDS + jj]
        .set(W1[:, :NPAIRS].T)
    )
    w1x = W1[:, NPAIRS:].T
    b1r = b1.reshape(1, HID)
    wout_t = Wout.T
    boutr = bout.reshape(1, 1)

    outs = []
    off = 0
    for bs in SLICES:
        nchunk = bs * FIELDS // (NWORKERS * CHUNK)
        idx3 = idx_flat[off * FIELDS : (off + bs) * FIELDS].reshape(
            NWORKERS, nchunk, CHUNK
        )
        embs3 = _sc_gather(tables_flat, idx3)
        outs.append(
            _tc_compute(embs3, xv[off : off + bs], w1g, w1x, b1r, wout_t, boutr)
        )
        off += bs
    return jnp.concatenate(outs, axis=0)
